# baseline (speedup 1.0000x reference)
import jax
import jax.numpy as jnp
from jax import lax
from jax.experimental import pallas as pl
from jax.experimental.pallas import tpu as pltpu
from jax.experimental.pallas import tpu_sc as plsc

N = 10000
E = 320000
IN = 15
H = 128
HH = H // 2
G = 256

NC = 2
NS = 16
NW = NC * NS
NPAD = 10112
SUBD = 82
SUBS = 2 * SUBD
EPAD = SUBD * 128 * NW
RPS = NPAD // NS

_mesh = plsc.VectorSubcoreMesh(core_axis_name="c", subcore_axis_name="s")
_f32 = jnp.float32


EROWS = E // 128


def _deg_body(ei3, ones_h, zeros16, degp, dstv, onesv, dacc):
    c = lax.axis_index("c")
    s = lax.axis_index("s")
    wid = c * NS + s
    base = 78 * wid + jnp.minimum(wid, 4)
    extra = wid < 4
    pltpu.sync_copy(ei3.at[1, pl.ds(base, 78)], dstv.at[pl.ds(0, 78)])

    @pl.when(extra)
    def _():
        pltpu.sync_copy(ei3.at[1, pl.ds(base + 78, 1)], dstv.at[pl.ds(78, 1)])

    pltpu.sync_copy(ones_h, onesv)
    pltpu.sync_copy(zeros16.at[pl.ds(s * RPS, RPS)], dacc.at[pl.ds(s * RPS, RPS)])
    plsc.subcore_barrier()

    def body(j, carry):
        pltpu.sync_copy(onesv, dacc.at[dstv.at[j]], add=True)
        return carry

    lax.fori_loop(0, 78, body, 0)

    @pl.when(extra)
    def _():
        pltpu.sync_copy(onesv, dacc.at[dstv.at[78]], add=True)

    plsc.subcore_barrier()
    pltpu.sync_copy(dacc.at[pl.ds(s * RPS, RPS)], degp.at[c, pl.ds(s * RPS, RPS)])


_deg = pl.kernel(
    _deg_body,
    out_type=jax.ShapeDtypeStruct((NC, NPAD, 16), _f32),
    mesh=_mesh,
    scratch_types=[
        pltpu.VMEM((79, 128), jnp.int32),
        pltpu.VMEM((128, 16), _f32),
        pltpu.VMEM_SHARED((NPAD, 16), _f32),
    ],
    compiler_params=pltpu.CompilerParams(use_tc_tiling_on_sc=False),
)


NBUF = 4


def _scat_body(srcb, dstb, ys, zeros_hh, accp, srcv, dstv, rows,
               g0, g1, g2, g3, s0, s1, s2, s3, acc):
    c = lax.axis_index("c")
    s = lax.axis_index("s")
    pltpu.sync_copy(srcb.at[c, s], srcv)
    pltpu.sync_copy(dstb.at[s], dstv)
    pltpu.sync_copy(zeros_hh.at[pl.ds(s * RPS, RPS)], acc.at[pl.ds(s * RPS, RPS)])
    plsc.subcore_barrier()

    gsem = (g0, g1, g2, g3)
    ssem = (s0, s1, s2, s3)

    def gather(j, b):
        pltpu.async_copy(ys.at[srcv.at[j]], rows.at[b], gsem[b])

    def wait_gather(j, b):
        pltpu.make_async_copy(ys.at[srcv.at[j]], rows.at[b], gsem[b]).wait()

    def scatter(j, b):
        pltpu.async_copy(rows.at[b], acc.at[dstv.at[j]], ssem[b], add=True)

    def wait_scatter(j, b):
        pltpu.make_async_copy(rows.at[b], acc.at[dstv.at[j]], ssem[b]).wait()

    def step(j, b, wait_s, fire_g):
        wait_gather(j, b)
        scatter(j, b)
        b2 = (b + 2) % NBUF
        if wait_s:
            wait_scatter(j - 2, b2)
        if fire_g:
            gather(j + 2, b2)

    gather(0, 0)
    gather(1, 1)
    step(0, 0, False, True)
    step(1, 1, False, True)

    def body(i, carry):
        for bb in range(NBUF):
            j = NBUF * i + 2 + bb
            step(j, (2 + bb) % NBUF, True, True)
        return carry

    lax.fori_loop(0, (SUBS - 4) // NBUF, body, 0)
    step(SUBS - 2, 2, True, False)
    step(SUBS - 1, 3, True, False)
    wait_scatter(SUBS - 2, 2)
    wait_scatter(SUBS - 1, 3)
    plsc.subcore_barrier()
    pltpu.sync_copy(acc.at[pl.ds(s * RPS, RPS)],
                    accp.at[pl.ds(s * RPS, RPS), pl.ds(c * HH, HH)])


_scat = pl.kernel(
    _scat_body,
    out_type=jax.ShapeDtypeStruct((NPAD, H), _f32),
    mesh=_mesh,
    scratch_types=[
        pltpu.VMEM((SUBS, 128), jnp.int32),
        pltpu.VMEM((SUBS, 128), jnp.int32),
        pltpu.VMEM((NBUF, 128, HH), _f32),
        pltpu.SemaphoreType.DMA,
        pltpu.SemaphoreType.DMA,
        pltpu.SemaphoreType.DMA,
        pltpu.SemaphoreType.DMA,
        pltpu.SemaphoreType.DMA,
        pltpu.SemaphoreType.DMA,
        pltpu.SemaphoreType.DMA,
        pltpu.SemaphoreType.DMA,
        pltpu.VMEM_SHARED((NPAD, HH), _f32),
    ],
    compiler_params=pltpu.CompilerParams(use_tc_tiling_on_sc=False),
)


NP8 = NPAD // 8
EP128 = EPAD // 128


def _prep_body(e3_ref, srcd_ref, dstd_ref):
    r = lax.broadcasted_iota(jnp.int32, (EP128, 128), 0)
    l = lax.broadcasted_iota(jnp.int32, (EP128, 128), 1)
    pos = r * 128 + l
    tail = jnp.where(pos < E + N, pos - E,
                     N + jnp.bitwise_and(pos - (E + N), 15))
    zpad = jnp.zeros((EP128 - E // 128, 128), jnp.int32)
    srcflat = jnp.where(pos < E,
                        jnp.concatenate([e3_ref[0], zpad], axis=0), tail)
    dstflat = jnp.where(pos < E,
                        jnp.concatenate([e3_ref[1], zpad], axis=0), tail)
    srcd_ref[0] = 2 * srcflat
    srcd_ref[1] = 2 * srcflat + 1
    dstd_ref[...] = dstflat


_prep = pl.pallas_call(
    _prep_body,
    out_shape=(
        jax.ShapeDtypeStruct((NC, EP128, 128), jnp.int32),
        jax.ShapeDtypeStruct((EP128, 128), jnp.int32),
    ),
)


def _dense0_body(x_ref, w_ref, xw_ref):
    xw = jnp.dot(x_ref[...], w_ref[...], preferred_element_type=_f32)
    xw_ref[...] = jnp.concatenate(
        [xw, jnp.zeros((NPAD - N, H), _f32)], axis=0)


_dense0 = pl.pallas_call(
    _dense0_body,
    out_shape=jax.ShapeDtypeStruct((NPAD, H), _f32),
)


def _sel8():
    ii = lax.broadcasted_iota(jnp.int32, (128, 8), 0)
    jj = lax.broadcasted_iota(jnp.int32, (128, 8), 1)
    return jnp.where(ii == 16 * jj, 1.0, 0.0).astype(_f32)


def _dense1_body(degp_ref, xw_ref, ys_ref, dinv_ref):
    sel = _sel8()
    deg8 = (jnp.dot(degp_ref[0], sel, preferred_element_type=_f32)
            + jnp.dot(degp_ref[1], sel, preferred_element_type=_f32) + 1.0)
    dinv8 = lax.rsqrt(deg8)
    xw3 = xw_ref[...].reshape(NP8, 8, H)
    ys_ref[...] = (xw3 * dinv8[:, :, None]).reshape(NPAD, H)
    dinv_ref[...] = dinv8


_dense1 = pl.pallas_call(
    _dense1_body,
    out_shape=(
        jax.ShapeDtypeStruct((NPAD, H), _f32),
        jax.ShapeDtypeStruct((NP8, 8), _f32),
    ),
)


def _dense2_body(acc_ref, dinv_ref, b_ref, w_ref, ys_ref):
    dinv8 = dinv_ref[...]
    acc3 = acc_ref[...].reshape(NP8, 8, H)
    h = jnp.maximum(acc3 * dinv8[:, :, None] + b_ref[...], 0.0)
    y = jnp.dot(h.reshape(NPAD, H), w_ref[...], preferred_element_type=_f32)
    ys_ref[...] = (y.reshape(NP8, 8, H) * dinv8[:, :, None]).reshape(NPAD, H)


_dense2 = pl.pallas_call(
    _dense2_body,
    out_shape=jax.ShapeDtypeStruct((NPAD, H), _f32),
)


def _pool_body(acc_ref, dinv_ref, b_ref, batch_ref, out_ref):
    dinv8 = dinv_ref[...]
    acc3 = acc_ref[...].reshape(NP8, 8, H)
    h = jnp.maximum(acc3 * dinv8[:, :, None] + b_ref[...], 0.0).reshape(NPAD, H)
    gids = lax.broadcasted_iota(jnp.int32, (G, NPAD), 0)
    onehot = jnp.where(gids == batch_ref[...], 1.0, 0.0)
    out_ref[...] = jnp.dot(onehot, h, preferred_element_type=_f32)


_pool = pl.pallas_call(
    _pool_body,
    out_shape=jax.ShapeDtypeStruct((G, H), _f32),
)


def kernel(x, edge_index, batch, W1, b1, W2, b2, W3, b3):
    ei3 = edge_index.reshape(2, E // 128, 128)
    srcd, dstd = _prep(ei3)
    srcb = srcd.reshape(NC, NS, SUBS, 128)
    dstb = dstd.reshape(NS, SUBS, 128)

    zeros_hh = jnp.zeros((NPAD, HH), _f32)
    zeros16 = jnp.zeros((NPAD, 16), _f32)
    ones16 = jnp.ones((128, 16), _f32)
    batchp = jnp.full((1, NPAD), G, jnp.int32).at[0, :N].set(batch)

    degp = _deg(ei3, ones16, zeros16)
    xw1 = _dense0(x, W1)
    y1, dinv8 = _dense1(degp.reshape(NC, NP8, 128), xw1)
    acc1 = _scat(srcb, dstb, y1.reshape(2 * NPAD, HH), zeros_hh)
    y2 = _dense2(acc1, dinv8, b1.reshape(1, 1, H), W2)
    acc2 = _scat(srcb, dstb, y2.reshape(2 * NPAD, HH), zeros_hh)
    y3 = _dense2(acc2, dinv8, b2.reshape(1, 1, H), W3)
    acc3 = _scat(srcb, dstb, y3.reshape(2 * NPAD, HH), zeros_hh)
    return _pool(acc3, dinv8, b3.reshape(1, 1, H), batchp)


# speedup vs baseline: 1.0074x; 1.0074x over previous
import jax
import jax.numpy as jnp
from jax import lax
from jax.experimental import pallas as pl
from jax.experimental.pallas import tpu as pltpu
from jax.experimental.pallas import tpu_sc as plsc

N = 10000
E = 320000
IN = 15
H = 128
HH = H // 2
G = 256

NC = 2
NS = 16
NW = NC * NS
NPAD = 10112
SUBD = 82
SUBS = 2 * SUBD
EPAD = SUBD * 128 * NW
RPS = NPAD // NS

_mesh = plsc.VectorSubcoreMesh(core_axis_name="c", subcore_axis_name="s")
_f32 = jnp.float32


def _deg_body(dst3, ones_h, zeros16, degp, dstv, onesv, dacc):
    c = lax.axis_index("c")
    s = lax.axis_index("s")
    wid = c * NS + s
    pltpu.sync_copy(dst3.at[wid], dstv)
    pltpu.sync_copy(ones_h, onesv)
    pltpu.sync_copy(zeros16.at[pl.ds(s * RPS, RPS)], dacc.at[pl.ds(s * RPS, RPS)])
    plsc.subcore_barrier()

    def body(j, carry):
        pltpu.sync_copy(onesv, dacc.at[dstv.at[j]], add=True)
        return carry

    lax.fori_loop(0, SUBD, body, 0)
    plsc.subcore_barrier()
    pltpu.sync_copy(dacc.at[pl.ds(s * RPS, RPS)], degp.at[c, pl.ds(s * RPS, RPS)])


_deg = pl.kernel(
    _deg_body,
    out_type=jax.ShapeDtypeStruct((NC, NPAD, 16), _f32),
    mesh=_mesh,
    scratch_types=[
        pltpu.VMEM((SUBD, 128), jnp.int32),
        pltpu.VMEM((128, 16), _f32),
        pltpu.VMEM_SHARED((NPAD, 16), _f32),
    ],
    compiler_params=pltpu.CompilerParams(use_tc_tiling_on_sc=False),
)


NBUF = 4


def _scat_body(srcb, dstb, ys, zeros_hh, accp, srcv, dstv, rows,
               g0, g1, g2, g3, s0, s1, s2, s3, acc):
    c = lax.axis_index("c")
    s = lax.axis_index("s")
    pltpu.sync_copy(srcb.at[c, s], srcv)
    pltpu.sync_copy(dstb.at[s], dstv)
    pltpu.sync_copy(zeros_hh.at[pl.ds(s * RPS, RPS)], acc.at[pl.ds(s * RPS, RPS)])
    plsc.subcore_barrier()

    gsem = (g0, g1, g2, g3)
    ssem = (s0, s1, s2, s3)

    def gather(j, b):
        pltpu.async_copy(ys.at[srcv.at[j]], rows.at[b], gsem[b])

    def wait_gather(j, b):
        pltpu.make_async_copy(ys.at[srcv.at[j]], rows.at[b], gsem[b]).wait()

    def scatter(j, b):
        pltpu.async_copy(rows.at[b], acc.at[dstv.at[j]], ssem[b], add=True)

    def wait_scatter(j, b):
        pltpu.make_async_copy(rows.at[b], acc.at[dstv.at[j]], ssem[b]).wait()

    def step(j, b, wait_s, fire_g):
        wait_gather(j, b)
        scatter(j, b)
        b2 = (b + 2) % NBUF
        if wait_s:
            wait_scatter(j - 2, b2)
        if fire_g:
            gather(j + 2, b2)

    gather(0, 0)
    gather(1, 1)
    step(0, 0, False, True)
    step(1, 1, False, True)

    def body(i, carry):
        for bb in range(NBUF):
            j = NBUF * i + 2 + bb
            step(j, (2 + bb) % NBUF, True, True)
        return carry

    lax.fori_loop(0, (SUBS - 4) // NBUF, body, 0)
    step(SUBS - 2, 2, True, False)
    step(SUBS - 1, 3, True, False)
    wait_scatter(SUBS - 2, 2)
    wait_scatter(SUBS - 1, 3)
    plsc.subcore_barrier()
    pltpu.sync_copy(acc.at[pl.ds(s * RPS, RPS)],
                    accp.at[pl.ds(s * RPS, RPS), pl.ds(c * HH, HH)])


_scat = pl.kernel(
    _scat_body,
    out_type=jax.ShapeDtypeStruct((NPAD, H), _f32),
    mesh=_mesh,
    scratch_types=[
        pltpu.VMEM((SUBS, 128), jnp.int32),
        pltpu.VMEM((SUBS, 128), jnp.int32),
        pltpu.VMEM((NBUF, 128, HH), _f32),
        pltpu.SemaphoreType.DMA,
        pltpu.SemaphoreType.DMA,
        pltpu.SemaphoreType.DMA,
        pltpu.SemaphoreType.DMA,
        pltpu.SemaphoreType.DMA,
        pltpu.SemaphoreType.DMA,
        pltpu.SemaphoreType.DMA,
        pltpu.SemaphoreType.DMA,
        pltpu.VMEM_SHARED((NPAD, HH), _f32),
    ],
    compiler_params=pltpu.CompilerParams(use_tc_tiling_on_sc=False),
)


NP8 = NPAD // 8
EP128 = EPAD // 128


def _prep_body(e3_ref, srcd_ref, dstd_ref):
    r = lax.broadcasted_iota(jnp.int32, (EP128, 128), 0)
    l = lax.broadcasted_iota(jnp.int32, (EP128, 128), 1)
    pos = r * 128 + l
    tail = jnp.where(pos < E + N, pos - E,
                     N + jnp.bitwise_and(pos - (E + N), 15))
    zpad = jnp.zeros((EP128 - E // 128, 128), jnp.int32)
    srcflat = jnp.where(pos < E,
                        jnp.concatenate([e3_ref[0], zpad], axis=0), tail)
    dstflat = jnp.where(pos < E,
                        jnp.concatenate([e3_ref[1], zpad], axis=0), tail)
    srcd_ref[0] = 2 * srcflat
    srcd_ref[1] = 2 * srcflat + 1
    dstd_ref[...] = dstflat


_prep = pl.pallas_call(
    _prep_body,
    out_shape=(
        jax.ShapeDtypeStruct((NC, EP128, 128), jnp.int32),
        jax.ShapeDtypeStruct((EP128, 128), jnp.int32),
    ),
)


def _dense0_body(x_ref, w_ref, xw_ref):
    xw = jnp.dot(x_ref[...], w_ref[...], preferred_element_type=_f32)
    xw_ref[...] = jnp.concatenate(
        [xw, jnp.zeros((NPAD - N, H), _f32)], axis=0)


_dense0 = pl.pallas_call(
    _dense0_body,
    out_shape=jax.ShapeDtypeStruct((NPAD, H), _f32),
)


def _sel8():
    ii = lax.broadcasted_iota(jnp.int32, (128, 8), 0)
    jj = lax.broadcasted_iota(jnp.int32, (128, 8), 1)
    return jnp.where(ii == 16 * jj, 1.0, 0.0).astype(_f32)


def _dense1_body(degp_ref, xw_ref, ys_ref, dinv_ref):
    sel = _sel8()
    deg8 = (jnp.dot(degp_ref[0], sel, preferred_element_type=_f32)
            + jnp.dot(degp_ref[1], sel, preferred_element_type=_f32))
    dinv8 = lax.rsqrt(jnp.maximum(deg8, 1.0))
    xw3 = xw_ref[...].reshape(NP8, 8, H)
    ys_ref[...] = (xw3 * dinv8[:, :, None]).reshape(NPAD, H)
    dinv_ref[...] = dinv8


_dense1 = pl.pallas_call(
    _dense1_body,
    out_shape=(
        jax.ShapeDtypeStruct((NPAD, H), _f32),
        jax.ShapeDtypeStruct((NP8, 8), _f32),
    ),
)


def _dense2_body(acc_ref, dinv_ref, b_ref, w_ref, ys_ref):
    rb = acc_ref.shape[0]
    dinv8 = dinv_ref[...]
    acc3 = acc_ref[...].reshape(rb // 8, 8, H)
    h = jnp.maximum(acc3 * dinv8[:, :, None] + b_ref[...], 0.0)
    y = jnp.dot(h.reshape(rb, H), w_ref[...], preferred_element_type=_f32)
    ys_ref[...] = (y.reshape(rb // 8, 8, H) * dinv8[:, :, None]).reshape(rb, H)


_dense2 = pl.pallas_call(
    _dense2_body,
    out_shape=jax.ShapeDtypeStruct((NPAD, H), _f32),
)


def _pool_body(acc_ref, dinv_ref, b_ref, batch_ref, out_ref):
    dinv8 = dinv_ref[...]
    acc3 = acc_ref[...].reshape(NP8, 8, H)
    h = jnp.maximum(acc3 * dinv8[:, :, None] + b_ref[...], 0.0).reshape(NPAD, H)
    gids = lax.broadcasted_iota(jnp.int32, (G, NPAD), 0)
    onehot = jnp.where(gids == batch_ref[...], 1.0, 0.0)
    out_ref[...] = jnp.dot(onehot, h, preferred_element_type=_f32)


_pool = pl.pallas_call(
    _pool_body,
    out_shape=jax.ShapeDtypeStruct((G, H), _f32),
)


def kernel(x, edge_index, batch, W1, b1, W2, b2, W3, b3):
    ei3 = edge_index.reshape(2, E // 128, 128)
    srcd, dstd = _prep(ei3)
    srcb = srcd.reshape(NC, NS, SUBS, 128)
    dstb = dstd.reshape(NS, SUBS, 128)
    dst3 = dstd.reshape(NW, SUBD, 128)

    zeros_hh = jnp.zeros((NPAD, HH), _f32)
    zeros16 = jnp.zeros((NPAD, 16), _f32)
    ones16 = jnp.ones((128, 16), _f32)
    batchp = jnp.full((1, NPAD), G, jnp.int32).at[0, :N].set(batch)

    degp = _deg(dst3, ones16, zeros16)
    xw1 = _dense0(x, W1)
    y1, dinv8 = _dense1(degp.reshape(NC, NP8, 128), xw1)
    acc1 = _scat(srcb, dstb, y1.reshape(2 * NPAD, HH), zeros_hh)
    y2 = _dense2(acc1, dinv8, b1.reshape(1, 1, H), W2)
    acc2 = _scat(srcb, dstb, y2.reshape(2 * NPAD, HH), zeros_hh)
    y3 = _dense2(acc2, dinv8, b2.reshape(1, 1, H), W3)
    acc3 = _scat(srcb, dstb, y3.reshape(2 * NPAD, HH), zeros_hh)
    return _pool(acc3, dinv8, b3.reshape(1, 1, H), batchp)
